# baseline (speedup 1.0000x reference)
import jax
import jax.numpy as jnp
from jax import lax
from jax.experimental import pallas as pl
from jax.experimental.pallas import tpu as pltpu
from jax.experimental.pallas import tpu_sc as plsc

NUM_EMB = 1000000
DIM = 32
BATCH = 16384
HIST = 50
NC = 2
NS = 16

CHUNK = 2048
CPH = BATCH // CHUNK
NCH = HIST * CPH
CPT = NCH // NS
IDX_RES = CPT * CHUNK
DPC = DIM // NC
SSEG = 62496
SREM = NUM_EMB - NS * SSEG


def _emb_body(idxT, wT, outT, idx_all, row_sh, gbuf, sem):
    cid = lax.axis_index("c")
    sid = lax.axis_index("s")

    def load_idx(j, carry):
        c = sid * CPT + j
        h = c // CPH
        b0 = (c % CPH) * CHUNK
        pltpu.sync_copy(idxT.at[h, pl.ds(b0, CHUNK)],
                        idx_all.at[pl.ds(j * CHUNK, CHUNK)])
        return carry

    lax.fori_loop(0, CPT, load_idx, 0)

    def per_d(dd, carry):
        d = cid * DPC + dd
        plsc.subcore_barrier()
        pltpu.sync_copy(wT.at[d, pl.ds(sid * SSEG, SSEG)],
                        row_sh.at[pl.ds(sid * SSEG, SSEG)])

        @pl.when(sid == 0)
        def _():
            pltpu.sync_copy(wT.at[d, pl.ds(NS * SSEG, SREM)],
                            row_sh.at[pl.ds(NS * SSEG, SREM)])

        plsc.subcore_barrier()

        def per_chunk(j, inner):
            c = sid * CPT + j
            h = c // CPH
            b0 = (c % CPH) * CHUNK
            pltpu.async_copy(row_sh.at[idx_all.at[pl.ds(j * CHUNK, CHUNK)]],
                             gbuf, sem).wait()
            pltpu.sync_copy(gbuf, outT.at[h, d, pl.ds(b0, CHUNK)])
            return inner

        lax.fori_loop(0, CPT, per_chunk, 0)
        return carry

    lax.fori_loop(0, DPC, per_d, 0)


def _embed(idxT, wT):
    mesh = plsc.VectorSubcoreMesh(core_axis_name="c", subcore_axis_name="s")
    return pl.kernel(
        _emb_body,
        mesh=mesh,
        out_type=jax.ShapeDtypeStruct((HIST, DIM, BATCH), jnp.float32),
        scratch_types=[
            pltpu.VMEM((IDX_RES,), jnp.int32),
            pltpu.VMEM_SHARED((NUM_EMB,), jnp.float32),
            pltpu.VMEM((CHUNK,), jnp.float32),
            pltpu.SemaphoreType.DMA,
        ],
        compiler_params=pltpu.CompilerParams(use_tc_tiling_on_sc=False),
    )(idxT, wT)


def kernel(input_, weight):
    idxT = input_.T.astype(jnp.int32)
    wT = lax.optimization_barrier(weight.T)
    outT = _embed(idxT, wT)
    return jnp.transpose(outT, (2, 0, 1))


# speedup vs baseline: 4.3859x; 4.3859x over previous
import jax
import jax.numpy as jnp
from jax import lax
from jax.experimental import pallas as pl
from jax.experimental.pallas import tpu as pltpu
from jax.experimental.pallas import tpu_sc as plsc

NUM_EMB = 1000000
DIM = 32
BATCH = 16384
HIST = 50
NC = 2
NS = 16

CHUNK = 2048
CPH = BATCH // CHUNK
NCH = HIST * CPH
CPT = NCH // NS
IDX_RES = CPT * CHUNK
DPC = DIM // NC

WMAIN = 999424
WTAIL = NUM_EMB - WMAIN
SSEG = WMAIN // NS
RCOL = WMAIN // 2


def _retile_body(src, dst, vbuf, sem):
    def band(b, carry):
        b0 = pl.multiple_of(b * 8, 8)
        for cc in range(2):
            c0 = cc * RCOL
            pltpu.sync_copy(src.at[pl.ds(b0, 8), pl.ds(c0, RCOL)], vbuf)
            for r in range(8):
                dst0 = (b * 8 + r) * WMAIN + c0
                pltpu.sync_copy(vbuf.at[r], dst.at[pl.ds(dst0, RCOL)])
        return carry

    lax.fori_loop(0, DIM // 8, band, 0)


def _retile(wTv):
    return pl.pallas_call(
        _retile_body,
        in_specs=[pl.BlockSpec(memory_space=pltpu.MemorySpace.HBM)],
        out_specs=pl.BlockSpec(memory_space=pltpu.MemorySpace.HBM),
        out_shape=jax.ShapeDtypeStruct((DIM * WMAIN,), jnp.float32),
        scratch_shapes=[
            pltpu.VMEM((8, RCOL), jnp.float32),
            pltpu.SemaphoreType.DMA,
        ],
    )(wTv)


def _emb_body(idxT, wmain, wtail, outT, idx_all, row_sh, gbuf, sem):
    cid = lax.axis_index("c")
    sid = lax.axis_index("s")

    def load_idx(j, carry):
        c = sid * CPT + j
        h = c // CPH
        b0 = (c % CPH) * CHUNK
        pltpu.sync_copy(idxT.at[h, pl.ds(b0, CHUNK)],
                        idx_all.at[pl.ds(j * CHUNK, CHUNK)])
        return carry

    lax.fori_loop(0, CPT, load_idx, 0)

    def per_d(dd, carry):
        d = cid * DPC + dd
        plsc.subcore_barrier()
        pltpu.sync_copy(wmain.at[d, pl.ds(sid * SSEG, SSEG)],
                        row_sh.at[pl.ds(sid * SSEG, SSEG)])

        @pl.when(sid == 0)
        def _():
            pltpu.sync_copy(wtail.at[d], row_sh.at[pl.ds(WMAIN, WTAIL)])

        plsc.subcore_barrier()

        def per_chunk(j, inner):
            c = sid * CPT + j
            h = c // CPH
            b0 = (c % CPH) * CHUNK
            pltpu.async_copy(row_sh.at[idx_all.at[pl.ds(j * CHUNK, CHUNK)]],
                             gbuf, sem).wait()
            pltpu.sync_copy(gbuf, outT.at[h, d, pl.ds(b0, CHUNK)])
            return inner

        lax.fori_loop(0, CPT, per_chunk, 0)
        return carry

    lax.fori_loop(0, DPC, per_d, 0)


def _embed(idxT, wmain, wtail):
    mesh = plsc.VectorSubcoreMesh(core_axis_name="c", subcore_axis_name="s")
    return pl.kernel(
        _emb_body,
        mesh=mesh,
        out_type=jax.ShapeDtypeStruct((HIST, DIM, BATCH), jnp.float32),
        scratch_types=[
            pltpu.VMEM((IDX_RES,), jnp.int32),
            pltpu.VMEM_SHARED((NUM_EMB,), jnp.float32),
            pltpu.VMEM((CHUNK,), jnp.float32),
            pltpu.SemaphoreType.DMA,
        ],
        compiler_params=pltpu.CompilerParams(use_tc_tiling_on_sc=False),
    )(idxT, wmain, wtail)


def kernel(input_, weight):
    idxT = input_.T.astype(jnp.int32)
    wmain = _retile(weight.T).reshape(DIM, WMAIN)
    wtail = weight[WMAIN:].T
    outT = _embed(idxT, wmain, wtail)
    return jnp.transpose(outT, (2, 0, 1))


# speedup vs baseline: 4.7467x; 1.0822x over previous
import jax
import jax.numpy as jnp
from jax import lax
from jax.experimental import pallas as pl
from jax.experimental.pallas import tpu as pltpu
from jax.experimental.pallas import tpu_sc as plsc

NUM_EMB = 1000000
DIM = 32
BATCH = 16384
HIST = 50
NC = 2
NS = 16

CHUNK = 2048
CPH = BATCH // CHUNK
NCH = HIST * CPH
CPT = NCH // NS
IDX_RES = CPT * CHUNK
DPC = DIM // NC
NB = 5

WMAIN = 999424
WTAIL = NUM_EMB - WMAIN
SSEG = WMAIN // NS
RCOL = WMAIN // 2


def _retile_body(src, dst, vbuf, sem):
    def band(b, carry):
        b0 = pl.multiple_of(b * 8, 8)
        for cc in range(2):
            c0 = cc * RCOL
            pltpu.sync_copy(src.at[pl.ds(b0, 8), pl.ds(c0, RCOL)], vbuf)
            for r in range(8):
                dst0 = (b * 8 + r) * WMAIN + c0
                pltpu.sync_copy(vbuf.at[r], dst.at[pl.ds(dst0, RCOL)])
        return carry

    lax.fori_loop(0, DIM // 8, band, 0)


def _retile(wTv):
    return pl.pallas_call(
        _retile_body,
        in_specs=[pl.BlockSpec(memory_space=pltpu.MemorySpace.HBM)],
        out_specs=pl.BlockSpec(memory_space=pltpu.MemorySpace.HBM),
        out_shape=jax.ShapeDtypeStruct((DIM * WMAIN,), jnp.float32),
        scratch_shapes=[
            pltpu.VMEM((8, RCOL), jnp.float32),
            pltpu.SemaphoreType.DMA,
        ],
    )(wTv)


def _emb_body(idxT, wmain, wtail, outT, idx_all, row_sh, pbuf, gsem, wsem):
    cid = lax.axis_index("c")
    sid = lax.axis_index("s")

    def load_idx(j, carry):
        c = sid * CPT + j
        h = c // CPH
        b0 = (c % CPH) * CHUNK
        pltpu.sync_copy(idxT.at[h, pl.ds(b0, CHUNK)],
                        idx_all.at[pl.ds(j * CHUNK, CHUNK)])
        return carry

    lax.fori_loop(0, CPT, load_idx, 0)

    def per_d(dd, carry):
        d = cid * DPC + dd
        plsc.subcore_barrier()
        pltpu.sync_copy(wmain.at[d, pl.ds(sid * SSEG, SSEG)],
                        row_sh.at[pl.ds(sid * SSEG, SSEG)])

        @pl.when(sid == 0)
        def _():
            pltpu.sync_copy(wtail.at[d], row_sh.at[pl.ds(WMAIN, WTAIL)])

        plsc.subcore_barrier()

        def grp(g, carry2):
            def fire_g(j, carry3):
                jj = g * NB + j
                pltpu.async_copy(
                    row_sh.at[idx_all.at[pl.ds(jj * CHUNK, CHUNK)]],
                    pbuf.at[pl.ds(j * CHUNK, CHUNK)], gsem)
                return carry3

            lax.fori_loop(0, NB, fire_g, 0)

            def drain_g(j, carry3):
                pltpu.make_async_copy(wmain.at[0, pl.ds(0, CHUNK)],
                                      pbuf.at[pl.ds(j * CHUNK, CHUNK)],
                                      gsem).wait()
                return carry3

            lax.fori_loop(0, NB, drain_g, 0)

            def fire_w(j, carry3):
                c = sid * CPT + g * NB + j
                h = c // CPH
                b0 = (c % CPH) * CHUNK
                pltpu.async_copy(pbuf.at[pl.ds(j * CHUNK, CHUNK)],
                                 outT.at[h, d, pl.ds(b0, CHUNK)], wsem)
                return carry3

            lax.fori_loop(0, NB, fire_w, 0)

            def drain_w(j, carry3):
                pltpu.make_async_copy(pbuf.at[pl.ds(j * CHUNK, CHUNK)],
                                      outT.at[0, 0, pl.ds(0, CHUNK)],
                                      wsem).wait()
                return carry3

            lax.fori_loop(0, NB, drain_w, 0)
            return carry2

        lax.fori_loop(0, CPT // NB, grp, 0)
        return carry

    lax.fori_loop(0, DPC, per_d, 0)


def _embed(idxT, wmain, wtail):
    mesh = plsc.VectorSubcoreMesh(core_axis_name="c", subcore_axis_name="s")
    return pl.kernel(
        _emb_body,
        mesh=mesh,
        out_type=jax.ShapeDtypeStruct((HIST, DIM, BATCH), jnp.float32),
        scratch_types=[
            pltpu.VMEM((IDX_RES,), jnp.int32),
            pltpu.VMEM_SHARED((NUM_EMB,), jnp.float32),
            pltpu.VMEM((NB * CHUNK,), jnp.float32),
            pltpu.SemaphoreType.DMA,
            pltpu.SemaphoreType.DMA,
        ],
        compiler_params=pltpu.CompilerParams(use_tc_tiling_on_sc=False),
    )(idxT, wmain, wtail)


def kernel(input_, weight):
    idxT = input_.T.astype(jnp.int32)
    wmain = _retile(weight.T).reshape(DIM, WMAIN)
    wtail = weight[WMAIN:].T
    outT = _embed(idxT, wmain, wtail)
    return jnp.transpose(outT, (2, 0, 1))
